# baseline (speedup 1.0000x reference)
import functools

import jax
import jax.numpy as jnp
from jax import lax
from jax.experimental import pallas as pl
from jax.experimental.pallas import tpu as pltpu
from jax.experimental.pallas import tpu_sc as plsc


def _sc_broadcast(table, b, s, d):
    info = plsc.get_sparse_core_info()
    nc, ns = info.num_cores, info.num_subcores
    nw = nc * ns
    rows_pw = s // nw
    chunk = 16
    n_chunks = rows_pw // chunk
    scale = float(d ** 0.5)

    nbuf = 6

    @functools.partial(
        pl.kernel,
        mesh=plsc.VectorSubcoreMesh(core_axis_name="c", subcore_axis_name="s"),
        out_type=jax.ShapeDtypeStruct((b, s, d), table.dtype),
        scratch_types=(
            [pltpu.VMEM((chunk, d), jnp.float32)] * nbuf
            + [pltpu.SemaphoreType.DMA] * (2 * nbuf)
        ),
    )
    def sc_k(table_hbm, out_hbm, *refs):
        bufs = refs[:nbuf]
        rsems = refs[nbuf:2 * nbuf]
        wsems = refs[2 * nbuf:]
        wid = lax.axis_index("s") * nc + lax.axis_index("c")
        base = wid * rows_pw

        def fire_read(ci):
            k = ci % nbuf
            return pltpu.async_copy(
                table_hbm.at[pl.ds(base + ci * chunk, chunk)], bufs[k], rsems[k])

        reads = {0: fire_read(0)}
        writes = {}
        for ci in range(n_chunks):
            k = ci % nbuf
            buf = bufs[k]
            reads.pop(ci).wait()
            nxt = ci + 1
            if nxt < n_chunks:
                for cp in writes.pop(nxt % nbuf, []):
                    cp.wait()
                reads[nxt] = fire_read(nxt)

            def sbody(r, _):
                for j in range(d // 16):
                    sl = (r, pl.ds(j * 16, 16))
                    buf[sl] = buf[sl] * scale
                return 0

            lax.fori_loop(0, chunk, sbody, 0)
            row0 = base + ci * chunk
            writes[k] = [
                pltpu.async_copy(buf, out_hbm.at[bi, pl.ds(row0, chunk)], wsems[k])
                for bi in range(b)
            ]
        for cps in writes.values():
            for cp in cps:
                cp.wait()

    return sc_k(table)


def kernel(x, table):
    b, s, d = x.shape
    return _sc_broadcast(table[:s], b, s, d)


# speedup vs baseline: 1.0490x; 1.0490x over previous
import functools

import jax
import jax.numpy as jnp
from jax import lax
from jax.experimental import pallas as pl
from jax.experimental.pallas import tpu as pltpu
from jax.experimental.pallas import tpu_sc as plsc


def _sc_broadcast(table, b, s, d):
    info = plsc.get_sparse_core_info()
    nc, ns = info.num_cores, info.num_subcores
    nw = nc * ns
    rows_pw = s // nw
    chunk = 56
    n_full = rows_pw // chunk
    chunks = [chunk] * n_full
    if rows_pw % chunk:
        chunks.append(rows_pw % chunk)
    n_chunks = len(chunks)
    scale = float(d ** 0.5)

    nbuf = 2

    @functools.partial(
        pl.kernel,
        mesh=plsc.VectorSubcoreMesh(core_axis_name="c", subcore_axis_name="s"),
        out_type=jax.ShapeDtypeStruct((b, s, d), table.dtype),
        scratch_types=(
            [pltpu.VMEM((chunk, d), jnp.float32)] * nbuf
            + [pltpu.SemaphoreType.DMA] * (2 * nbuf)
        ),
    )
    def sc_k(table_hbm, out_hbm, *refs):
        bufs = refs[:nbuf]
        rsems = refs[nbuf:2 * nbuf]
        wsems = refs[2 * nbuf:]
        wid = lax.axis_index("s") * nc + lax.axis_index("c")
        base = wid * rows_pw
        starts = [sum(chunks[:i]) for i in range(n_chunks)]

        def fire_read(ci):
            k = ci % nbuf
            c = chunks[ci]
            dst = bufs[k] if c == chunk else bufs[k].at[pl.ds(0, c)]
            return pltpu.async_copy(
                table_hbm.at[pl.ds(base + starts[ci], c)], dst, rsems[k])

        reads = {0: fire_read(0)}
        writes = {}
        for ci in range(n_chunks):
            k = ci % nbuf
            c = chunks[ci]
            buf = bufs[k]
            reads.pop(ci).wait()
            nxt = ci + 1
            if nxt < n_chunks:
                for cp in writes.pop(nxt % nbuf, []):
                    cp.wait()
                reads[nxt] = fire_read(nxt)

            def sbody(r, _):
                for j in range(d // 16):
                    sl = (r, pl.ds(j * 16, 16))
                    buf[sl] = buf[sl] * scale
                return 0

            lax.fori_loop(0, c, sbody, 0)
            row0 = base + starts[ci]
            src = buf if c == chunk else buf.at[pl.ds(0, c)]
            writes[k] = [
                pltpu.async_copy(src, out_hbm.at[bi, pl.ds(row0, c)], wsems[k])
                for bi in range(b)
            ]
        for cps in writes.values():
            for cp in cps:
                cp.wait()

    return sc_k(table)


def kernel(x, table):
    b, s, d = x.shape
    return _sc_broadcast(table[:s], b, s, d)


# speedup vs baseline: 1.0762x; 1.0259x over previous
import functools

import jax
import jax.numpy as jnp
from jax import lax
from jax.experimental import pallas as pl
from jax.experimental.pallas import tpu as pltpu
from jax.experimental.pallas import tpu_sc as plsc


def _sc_broadcast(table, b, s, d):
    info = plsc.get_sparse_core_info()
    nc, ns = info.num_cores, info.num_subcores
    nw = nc * ns
    rows_pw = s // nw
    chunk = 40
    n_full = rows_pw // chunk
    chunks = [chunk] * n_full
    if rows_pw % chunk:
        chunks.append(rows_pw % chunk)
    n_chunks = len(chunks)
    scale = float(d ** 0.5)

    nbuf = 3

    @functools.partial(
        pl.kernel,
        mesh=plsc.VectorSubcoreMesh(core_axis_name="c", subcore_axis_name="s"),
        out_type=jax.ShapeDtypeStruct((b, s, d), table.dtype),
        scratch_types=(
            [pltpu.VMEM((chunk, d), jnp.float32)] * nbuf
            + [pltpu.SemaphoreType.DMA] * (2 * nbuf)
        ),
    )
    def sc_k(table_hbm, out_hbm, *refs):
        bufs = refs[:nbuf]
        rsems = refs[nbuf:2 * nbuf]
        wsems = refs[2 * nbuf:]
        wid = lax.axis_index("s") * nc + lax.axis_index("c")
        base = wid * rows_pw
        starts = [sum(chunks[:i]) for i in range(n_chunks)]

        def fire_read(ci):
            k = ci % nbuf
            c = chunks[ci]
            dst = bufs[k] if c == chunk else bufs[k].at[pl.ds(0, c)]
            return pltpu.async_copy(
                table_hbm.at[pl.ds(base + starts[ci], c)], dst, rsems[k])

        reads = {0: fire_read(0)}
        writes = {}
        for ci in range(n_chunks):
            k = ci % nbuf
            c = chunks[ci]
            buf = bufs[k]
            reads.pop(ci).wait()
            nxt = ci + 1
            if nxt < n_chunks:
                for cp in writes.pop(nxt % nbuf, []):
                    cp.wait()
                reads[nxt] = fire_read(nxt)

            def sbody(r, _):
                for j in range(d // 16):
                    sl = (r, pl.ds(j * 16, 16))
                    buf[sl] = buf[sl] * scale
                return 0

            lax.fori_loop(0, c, sbody, 0)
            row0 = base + starts[ci]
            src = buf if c == chunk else buf.at[pl.ds(0, c)]
            writes[k] = [
                pltpu.async_copy(src, out_hbm.at[bi, pl.ds(row0, c)], wsems[k])
                for bi in range(b)
            ]
        for cps in writes.values():
            for cp in cps:
                cp.wait()

    return sc_k(table)


def kernel(x, table):
    b, s, d = x.shape
    return _sc_broadcast(table[:s], b, s, d)
